# baseline (speedup 1.0000x reference)
import dataclasses
import functools

import jax
import jax.numpy as jnp
from jax import lax
from jax.experimental import pallas as pl
from jax.experimental.pallas import tpu as pltpu
from jax.experimental.pallas import tpu_sc as plsc

N = 10000
E = 320000
D = 128
H = 128
B = 128
C = 10

NC = 2
NS = 16
NW = NC * NS
EK = 128
NBLKP = 2560
BPW = NBLKP // NW
EPAD = NBLKP * EK - E
NP = 10240
RPT = NP // NS
ROWS_W = 320
PCH = 16
MBLK = 400
NPAD = 10112

_F32 = jnp.float32
_NEG = -3.0e38

_vec_mesh = plsc.VectorSubcoreMesh(core_axis_name="c", subcore_axis_name="s")

_sc_params = pltpu.CompilerParams()
if "needs_layout_passes" in pltpu.CompilerParams.__dataclass_fields__:
    _sc_params = dataclasses.replace(_sc_params, needs_layout_passes=False)


def _wid():
    return lax.axis_index("s") * NC + lax.axis_index("c")


def _fill(ref, nrows, vec16):
    @pl.loop(0, nrows)
    def _(i):
        for k in range(8):
            ref[i, pl.ds(16 * k, 16)] = vec16


@functools.partial(
    pl.kernel,
    mesh=_vec_mesh,
    out_type=jax.ShapeDtypeStruct((2 * NP, H), _F32),
    scratch_types=[
        pltpu.VMEM_SHARED((NP, H), _F32),
        pltpu.VMEM((EK, H), _F32),
        [pltpu.VMEM((EK,), jnp.int32)] * 4,
        [pltpu.SemaphoreType.DMA] * 4,
        [pltpu.SemaphoreType.DMA] * 2,
    ],
)
def _sc_deg(dstp_hbm, out_hbm, acc_sh, ones_v, dvs, sis, sss):
    c = lax.axis_index("c")
    s = lax.axis_index("s")
    w = _wid()

    _fill(ones_v, EK, jnp.zeros((16,), _F32))

    @pl.loop(0, 5)
    def _(t):
        pltpu.sync_copy(ones_v, acc_sh.at[pl.ds(s * RPT + t * EK, EK)])

    _fill(ones_v, EK, jnp.ones((16,), _F32))

    def idx_slice(t):
        return dstp_hbm.at[pl.ds((w * BPW + t) * EK, EK)]

    for t in range(2):
        pltpu.async_copy(idx_slice(t), dvs[t], sis[t])

    plsc.subcore_barrier()

    @pl.loop(0, BPW // 4)
    def _(i):
        for b in range(4):
            t = 4 * i + b

            @pl.when(t >= 2)
            def _(t=t, b=b):
                pltpu.make_async_copy(
                    ones_v, acc_sh.at[dvs[(b + 2) % 4]], sss[b % 2]).wait()

            pltpu.make_async_copy(idx_slice(t), dvs[b], sis[b]).wait()

            @pl.when(t < BPW - 2)
            def _(t=t, b=b):
                pltpu.async_copy(
                    idx_slice(t + 2), dvs[(b + 2) % 4], sis[(b + 2) % 4])

            pltpu.async_copy(ones_v, acc_sh.at[dvs[b]], sss[b % 2], add=True)

    for b in range(2):
        pltpu.make_async_copy(
            ones_v, acc_sh.at[dvs[(BPW - 2 + b) % 4]], sss[b]).wait()

    plsc.subcore_barrier()
    pltpu.sync_copy(acc_sh.at[pl.ds(s * RPT, RPT)],
                    out_hbm.at[pl.ds(c * NP + s * RPT, RPT)])


@functools.partial(
    pl.kernel,
    mesh=_vec_mesh,
    out_type=jax.ShapeDtypeStruct((2 * NP, H), _F32),
    scratch_types=[
        pltpu.VMEM_SHARED((NP, H), _F32),
        [pltpu.VMEM((EK, H), _F32)] * 2,
        [pltpu.VMEM((EK,), jnp.int32)] * 4,
        [pltpu.VMEM((EK,), jnp.int32)] * 4,
        [pltpu.SemaphoreType.DMA] * 4,
        [pltpu.SemaphoreType.DMA] * 4,
        [pltpu.SemaphoreType.DMA] * 2,
    ],
)
def _sc_conv(g_hbm, srcp_hbm, dstp_hbm, out_hbm, acc_sh, rows, svs, dvs,
             sis, sjs, sgs):
    c = lax.axis_index("c")
    s = lax.axis_index("s")
    w = _wid()

    _fill(rows[0], EK, jnp.zeros((16,), _F32))

    @pl.loop(0, 5)
    def _(t):
        pltpu.sync_copy(rows[0], acc_sh.at[pl.ds(s * RPT + t * EK, EK)])

    def src_slice(t):
        return srcp_hbm.at[pl.ds((w * BPW + t) * EK, EK)]

    def dst_slice(t):
        return dstp_hbm.at[pl.ds((w * BPW + t) * EK, EK)]

    plsc.subcore_barrier()

    @pl.loop(0, BPW)
    def _(t):
        pltpu.sync_copy(src_slice(t), svs[0])
        pltpu.sync_copy(dst_slice(t), dvs[0])
        pltpu.sync_copy(g_hbm.at[svs[0]], rows[0])
        pltpu.sync_copy(rows[0], acc_sh.at[dvs[0]], add=True)

    plsc.subcore_barrier()
    pltpu.sync_copy(acc_sh.at[pl.ds(s * RPT, RPT)],
                    out_hbm.at[pl.ds(c * NP + s * RPT, RPT)])


@functools.partial(
    pl.kernel,
    mesh=_vec_mesh,
    out_type=jax.ShapeDtypeStruct((NW, B, 2 * H), _F32),
    scratch_types=[
        pltpu.VMEM((B, 2 * H), _F32),
        pltpu.VMEM((PCH, H), _F32),
        pltpu.VMEM((PCH,), jnp.int32),
    ],
    compiler_params=_sc_params,
)
def _sc_pool(h_hbm, batch_hbm, out_hbm, acc_v, h_v, b_v):
    w = _wid()
    zero16 = jnp.zeros((16,), _F32)
    neg16 = jnp.full((16,), _NEG, _F32)
    iota16 = lax.broadcasted_iota(jnp.int32, (16,), 0)

    @pl.loop(0, B)
    def _(i):
        for k in range(8):
            acc_v[i, pl.ds(16 * k, 16)] = zero16
            acc_v[i, pl.ds(H + 16 * k, 16)] = neg16

    lo = w * ROWS_W
    nchunks = jnp.minimum(ROWS_W, N - lo) // PCH

    def chunk(t, carry):
        off = lo + t * PCH
        pltpu.sync_copy(batch_hbm.at[pl.ds(off, PCH)], b_v)
        pltpu.sync_copy(h_hbm.at[pl.ds(off, PCH)], h_v)
        bvec = b_v[...]
        for r in range(PCH):
            b = jnp.max(jnp.where(iota16 == r, bvec, 0))
            for k in range(8):
                hx = h_v[r, pl.ds(16 * k, 16)]
                acc_v[b, pl.ds(16 * k, 16)] = acc_v[b, pl.ds(16 * k, 16)] + hx
                acc_v[b, pl.ds(H + 16 * k, 16)] = jnp.maximum(
                    acc_v[b, pl.ds(H + 16 * k, 16)], hx)
        return carry

    lax.fori_loop(0, nchunks, chunk, 0)
    pltpu.sync_copy(acc_v, out_hbm.at[w])


def _prep_body(p0, p1, x, w, dinv_o, z_o, g_o):
    deg = 1.0 + p0[:, 0:1] + p1[:, 0:1]
    dinv = lax.rsqrt(deg)
    z = jnp.dot(x[...], w[...], preferred_element_type=_F32)
    dinv_o[...] = dinv
    z_o[...] = z
    g_o[...] = dinv * z


def _tc_prep(p0, p1, x, w1):
    return pl.pallas_call(
        _prep_body,
        grid=(N // MBLK,),
        in_specs=[
            pl.BlockSpec((MBLK, H), lambda i: (i, 0)),
            pl.BlockSpec((MBLK, H), lambda i: (i, 0)),
            pl.BlockSpec((MBLK, D), lambda i: (i, 0)),
            pl.BlockSpec((D, H), lambda i: (0, 0)),
        ],
        out_specs=[
            pl.BlockSpec((MBLK, 1), lambda i: (i, 0)),
            pl.BlockSpec((MBLK, H), lambda i: (i, 0)),
            pl.BlockSpec((MBLK, H), lambda i: (i, 0)),
        ],
        out_shape=[
            jax.ShapeDtypeStruct((N, 1), _F32),
            jax.ShapeDtypeStruct((N, H), _F32),
            jax.ShapeDtypeStruct((N, H), _F32),
        ],
    )(p0, p1, x, w1)


def _mid_body(q0, q1, z, dinv, b, w, h_o, zn_o, gn_o):
    dv = dinv[...]
    h = jnp.maximum(dv * (q0[...] + q1[...]) + dv * dv * z[...] + b[...], 0.0)
    h_o[...] = h
    zn = jnp.dot(h, w[...], preferred_element_type=_F32)
    zn_o[...] = zn
    gn_o[...] = dv * zn


def _tc_mid(q0, q1, z, dinv, b, wnext):
    return pl.pallas_call(
        _mid_body,
        grid=(N // MBLK,),
        in_specs=[
            pl.BlockSpec((MBLK, H), lambda i: (i, 0)),
            pl.BlockSpec((MBLK, H), lambda i: (i, 0)),
            pl.BlockSpec((MBLK, H), lambda i: (i, 0)),
            pl.BlockSpec((MBLK, 1), lambda i: (i, 0)),
            pl.BlockSpec((1, H), lambda i: (0, 0)),
            pl.BlockSpec((H, H), lambda i: (0, 0)),
        ],
        out_specs=[
            pl.BlockSpec((MBLK, H), lambda i: (i, 0)),
            pl.BlockSpec((MBLK, H), lambda i: (i, 0)),
            pl.BlockSpec((MBLK, H), lambda i: (i, 0)),
        ],
        out_shape=[
            jax.ShapeDtypeStruct((N, H), _F32),
            jax.ShapeDtypeStruct((N, H), _F32),
            jax.ShapeDtypeStruct((N, H), _F32),
        ],
    )(q0, q1, z, dinv, b, wnext)


def _last_body(q0, q1, z, dinv, b, h_o):
    dv = dinv[...]
    h_o[...] = jnp.maximum(
        dv * (q0[...] + q1[...]) + dv * dv * z[...] + b[...], 0.0)


def _tc_last(q0, q1, z, dinv, b):
    return pl.pallas_call(
        _last_body,
        grid=(N // MBLK,),
        in_specs=[
            pl.BlockSpec((MBLK, H), lambda i: (i, 0)),
            pl.BlockSpec((MBLK, H), lambda i: (i, 0)),
            pl.BlockSpec((MBLK, H), lambda i: (i, 0)),
            pl.BlockSpec((MBLK, 1), lambda i: (i, 0)),
            pl.BlockSpec((1, H), lambda i: (0, 0)),
        ],
        out_specs=pl.BlockSpec((MBLK, H), lambda i: (i, 0)),
        out_shape=jax.ShapeDtypeStruct((N, H), _F32),
    )(q0, q1, z, dinv, b)


def _fin_body(pt1, pt2, pt3, batchr, fw1, fb1, fw2, fb2, fw3, fb3,
              out_o, a1, a2, a3):
    w = pl.program_id(0)
    for pt, a in ((pt1, a1), (pt2, a2), (pt3, a3)):
        blk = pt[0]

        @pl.when(w == 0)
        def _(blk=blk, a=a):
            a[...] = blk

        @pl.when(w > 0)
        def _(blk=blk, a=a):
            s = a[:, :H] + blk[:, :H]
            m = jnp.maximum(a[:, H:], blk[:, H:])
            a[...] = jnp.concatenate([s, m], axis=1)

    @pl.when(w == NW - 1)
    def _():
        bv = batchr[...]
        iota_c = lax.broadcasted_iota(jnp.int32, (B, NPAD), 0)
        onehot = (bv == iota_c).astype(_F32)
        cnt = jnp.sum(onehot, axis=1, keepdims=True)
        invc = jnp.where(cnt > 0, 1.0 / jnp.maximum(cnt, 1.0), 0.0)
        zs = jnp.zeros((B, 3 * H), _F32)
        for a in (a1, a2, a3):
            sacc = a[:, :H]
            macc = jnp.where(cnt > 0, a[:, H:], 0.0)
            zs = zs + jnp.concatenate([sacc * invc, macc, sacc], axis=1)
        t1 = jnp.maximum(
            jnp.dot(zs, fw1[...], preferred_element_type=_F32) + fb1[...], 0.0)
        t2 = jnp.maximum(
            jnp.dot(t1, fw2[...], preferred_element_type=_F32) + fb2[...], 0.0)
        logits = jnp.dot(t2, fw3[...], preferred_element_type=_F32) + fb3[...]
        mxl = jnp.max(logits, axis=-1, keepdims=True)
        ex = jnp.exp(logits - mxl)
        lse = mxl + jnp.log(jnp.sum(ex, axis=-1, keepdims=True))
        out_o[...] = logits - lse


def _tc_final(pt1, pt2, pt3, batchr, fw1, fb1, fw2, fb2, fw3, fb3):
    return pl.pallas_call(
        _fin_body,
        grid=(NW,),
        in_specs=[
            pl.BlockSpec((1, B, 2 * H), lambda w: (w, 0, 0)),
            pl.BlockSpec((1, B, 2 * H), lambda w: (w, 0, 0)),
            pl.BlockSpec((1, B, 2 * H), lambda w: (w, 0, 0)),
            pl.BlockSpec((1, NPAD), lambda w: (0, 0)),
            pl.BlockSpec((3 * H, H), lambda w: (0, 0)),
            pl.BlockSpec((1, H), lambda w: (0, 0)),
            pl.BlockSpec((H, H // 2), lambda w: (0, 0)),
            pl.BlockSpec((1, H // 2), lambda w: (0, 0)),
            pl.BlockSpec((H // 2, 128), lambda w: (0, 0)),
            pl.BlockSpec((1, 128), lambda w: (0, 0)),
        ],
        out_specs=pl.BlockSpec((B, 128), lambda w: (0, 0)),
        out_shape=jax.ShapeDtypeStruct((B, 128), _F32),
        scratch_shapes=[
            pltpu.VMEM((B, 2 * H), _F32),
            pltpu.VMEM((B, 2 * H), _F32),
            pltpu.VMEM((B, 2 * H), _F32),
        ],
    )(pt1, pt2, pt3, batchr, fw1, fb1, fw2, fb2, fw3, fb3)


@jax.jit
def kernel(x, edge_index, batch, y, W1, b1, W2, b2, W3, b3,
           fW1, fb1, fW2, fb2, fW3, fb3):
    src = edge_index[0].astype(jnp.int32)
    dst = edge_index[1].astype(jnp.int32)
    batch = batch.astype(jnp.int32)

    srcp = jnp.concatenate([src, jnp.zeros((EPAD,), jnp.int32)])
    dstp = jnp.concatenate(
        [dst, N + jnp.arange(EPAD, dtype=jnp.int32) % (NP - N)])

    degp = _sc_deg(dstp)
    dinv, z1, g1 = _tc_prep(degp[:N], degp[NP:NP + N], x, W1)

    q1 = _sc_conv(g1, srcp, dstp)
    h1, z2, g2 = _tc_mid(q1[:N], q1[NP:NP + N], z1, dinv, b1.reshape(1, H), W2)

    q2 = _sc_conv(g2, srcp, dstp)
    h2, z3, g3 = _tc_mid(q2[:N], q2[NP:NP + N], z2, dinv, b2.reshape(1, H), W3)

    q3 = _sc_conv(g3, srcp, dstp)
    h3 = _tc_last(q3[:N], q3[NP:NP + N], z3, dinv, b3.reshape(1, H))

    p1 = _sc_pool(h1, batch)
    p2 = _sc_pool(h2, batch)
    p3 = _sc_pool(h3, batch)

    batchr = jnp.concatenate(
        [batch, jnp.full((NPAD - N,), B, jnp.int32)]).reshape(1, NPAD)
    fw3p = jnp.pad(fW3, ((0, 0), (0, 128 - C)))
    fb3p = jnp.pad(fb3, (0, 128 - C), constant_values=_NEG).reshape(1, 128)

    out = _tc_final(p1, p2, p3, batchr,
                    fW1, fb1.reshape(1, H),
                    fW2, fb2.reshape(1, H // 2),
                    fw3p, fb3p)
    return out[:, :C]


# speedup vs baseline: 3.0510x; 3.0510x over previous
import dataclasses
import functools

import jax
import jax.numpy as jnp
from jax import lax
from jax.experimental import pallas as pl
from jax.experimental.pallas import tpu as pltpu
from jax.experimental.pallas import tpu_sc as plsc

N = 10000
E = 320000
D = 128
H = 128
B = 128
C = 10

NC = 2
NS = 16
NW = NC * NS
EK = 128
NBLKP = 2560
BPW = NBLKP // NW
EPAD = NBLKP * EK - E
NP = 10240
RPT = NP // NS
ROWS_W = 320
PCH = 16
MBLK = 400
NPAD = 10112

_F32 = jnp.float32
_NEG = -3.0e38

_vec_mesh = plsc.VectorSubcoreMesh(core_axis_name="c", subcore_axis_name="s")

_sc_params = pltpu.CompilerParams()
if "needs_layout_passes" in pltpu.CompilerParams.__dataclass_fields__:
    _sc_params = dataclasses.replace(_sc_params, needs_layout_passes=False)


def _wid():
    return lax.axis_index("s") * NC + lax.axis_index("c")


def _fill(ref, nrows, vec16):
    @pl.loop(0, nrows)
    def _(i):
        for k in range(8):
            ref[i, pl.ds(16 * k, 16)] = vec16


@functools.partial(
    pl.kernel,
    mesh=_vec_mesh,
    out_type=jax.ShapeDtypeStruct((2 * NP, H), _F32),
    scratch_types=[
        pltpu.VMEM_SHARED((NP, H), _F32),
        pltpu.VMEM((EK, H), _F32),
        [pltpu.VMEM((EK,), jnp.int32)] * 4,
        [pltpu.SemaphoreType.DMA] * 4,
        [pltpu.SemaphoreType.DMA] * 2,
    ],
)
def _sc_deg(dstp_hbm, out_hbm, acc_sh, ones_v, dvs, sis, sss):
    c = lax.axis_index("c")
    s = lax.axis_index("s")
    w = _wid()

    _fill(ones_v, EK, jnp.zeros((16,), _F32))

    @pl.loop(0, 5)
    def _(t):
        pltpu.sync_copy(ones_v, acc_sh.at[pl.ds(s * RPT + t * EK, EK)])

    _fill(ones_v, EK, jnp.ones((16,), _F32))

    def idx_slice(t):
        return dstp_hbm.at[pl.ds((w * BPW + t) * EK, EK)]

    for t in range(2):
        pltpu.async_copy(idx_slice(t), dvs[t], sis[t])

    plsc.subcore_barrier()

    @pl.loop(0, BPW // 4)
    def _(i):
        for b in range(4):
            t = 4 * i + b

            @pl.when(t >= 2)
            def _(t=t, b=b):
                pltpu.make_async_copy(
                    ones_v, acc_sh.at[dvs[(b + 2) % 4]], sss[b % 2]).wait()

            pltpu.make_async_copy(idx_slice(t), dvs[b], sis[b]).wait()

            @pl.when(t < BPW - 2)
            def _(t=t, b=b):
                pltpu.async_copy(
                    idx_slice(t + 2), dvs[(b + 2) % 4], sis[(b + 2) % 4])

            pltpu.async_copy(ones_v, acc_sh.at[dvs[b]], sss[b % 2], add=True)

    for b in range(2):
        pltpu.make_async_copy(
            ones_v, acc_sh.at[dvs[(BPW - 2 + b) % 4]], sss[b]).wait()

    plsc.subcore_barrier()
    pltpu.sync_copy(acc_sh.at[pl.ds(s * RPT, RPT)],
                    out_hbm.at[pl.ds(c * NP + s * RPT, RPT)])


@functools.partial(
    pl.kernel,
    mesh=_vec_mesh,
    out_type=jax.ShapeDtypeStruct((2 * NP, H), _F32),
    scratch_types=[
        pltpu.VMEM_SHARED((NP, H), _F32),
        [pltpu.VMEM((EK, H), _F32)] * 2,
        [pltpu.VMEM((EK,), jnp.int32)] * 4,
        [pltpu.VMEM((EK,), jnp.int32)] * 4,
        [pltpu.SemaphoreType.DMA] * 4,
        [pltpu.SemaphoreType.DMA] * 4,
        [pltpu.SemaphoreType.DMA] * 2,
    ],
)
def _sc_conv(g_hbm, srcp_hbm, dstp_hbm, out_hbm, acc_sh, rows, svs, dvs,
             sis, sjs, sgs):
    c = lax.axis_index("c")
    s = lax.axis_index("s")
    w = _wid()

    _fill(rows[0], EK, jnp.zeros((16,), _F32))

    @pl.loop(0, 5)
    def _(t):
        pltpu.sync_copy(rows[0], acc_sh.at[pl.ds(s * RPT + t * EK, EK)])

    def src_slice(t):
        return srcp_hbm.at[pl.ds((w * BPW + t) * EK, EK)]

    def dst_slice(t):
        return dstp_hbm.at[pl.ds((w * BPW + t) * EK, EK)]

    for t in range(2):
        pltpu.async_copy(src_slice(t), svs[t], sis[t])
        pltpu.async_copy(dst_slice(t), dvs[t], sjs[t])
    pltpu.make_async_copy(src_slice(0), svs[0], sis[0]).wait()
    pltpu.async_copy(g_hbm.at[svs[0]], rows[0], sgs[0])

    plsc.subcore_barrier()

    @pl.loop(0, BPW // 4)
    def _(i):
        for b in range(4):
            t = 4 * i + b
            b2 = b % 2

            pltpu.make_async_copy(g_hbm.at[svs[b]], rows[b2], sgs[b2]).wait()

            @pl.when(t < BPW - 1)
            def _(t=t, b=b, b2=b2):
                nb = (b + 1) % 4
                pltpu.make_async_copy(src_slice(t + 1), svs[nb], sis[nb]).wait()
                pltpu.async_copy(g_hbm.at[svs[nb]], rows[1 - b2], sgs[1 - b2])

            @pl.when(t < BPW - 2)
            def _(t=t, b=b):
                nb = (b + 2) % 4
                pltpu.async_copy(src_slice(t + 2), svs[nb], sis[nb])
                pltpu.async_copy(dst_slice(t + 2), dvs[nb], sjs[nb])

            pltpu.make_async_copy(dst_slice(t), dvs[b], sjs[b]).wait()
            pltpu.sync_copy(rows[b2], acc_sh.at[dvs[b]], add=True)

    plsc.subcore_barrier()
    pltpu.sync_copy(acc_sh.at[pl.ds(s * RPT, RPT)],
                    out_hbm.at[pl.ds(c * NP + s * RPT, RPT)])


@functools.partial(
    pl.kernel,
    mesh=_vec_mesh,
    out_type=jax.ShapeDtypeStruct((NW, B, 2 * H), _F32),
    scratch_types=[
        pltpu.VMEM((B, 2 * H), _F32),
        pltpu.VMEM((PCH, H), _F32),
        pltpu.VMEM((PCH,), jnp.int32),
    ],
    compiler_params=_sc_params,
)
def _sc_pool(h_hbm, batch_hbm, out_hbm, acc_v, h_v, b_v):
    w = _wid()
    zero16 = jnp.zeros((16,), _F32)
    neg16 = jnp.full((16,), _NEG, _F32)
    iota16 = lax.broadcasted_iota(jnp.int32, (16,), 0)

    @pl.loop(0, B)
    def _(i):
        for k in range(8):
            acc_v[i, pl.ds(16 * k, 16)] = zero16
            acc_v[i, pl.ds(H + 16 * k, 16)] = neg16

    lo = w * ROWS_W
    nchunks = jnp.minimum(ROWS_W, N - lo) // PCH

    def chunk(t, carry):
        off = lo + t * PCH
        pltpu.sync_copy(batch_hbm.at[pl.ds(off, PCH)], b_v)
        pltpu.sync_copy(h_hbm.at[pl.ds(off, PCH)], h_v)
        bvec = b_v[...]
        for r in range(PCH):
            b = jnp.max(jnp.where(iota16 == r, bvec, 0))
            for k in range(8):
                hx = h_v[r, pl.ds(16 * k, 16)]
                acc_v[b, pl.ds(16 * k, 16)] = acc_v[b, pl.ds(16 * k, 16)] + hx
                acc_v[b, pl.ds(H + 16 * k, 16)] = jnp.maximum(
                    acc_v[b, pl.ds(H + 16 * k, 16)], hx)
        return carry

    lax.fori_loop(0, nchunks, chunk, 0)
    pltpu.sync_copy(acc_v, out_hbm.at[w])


def _prep_body(p0, p1, x, w, dinv_o, z_o, g_o):
    deg = 1.0 + p0[:, 0:1] + p1[:, 0:1]
    dinv = lax.rsqrt(deg)
    z = jnp.dot(x[...], w[...], preferred_element_type=_F32)
    dinv_o[...] = dinv
    z_o[...] = z
    g_o[...] = dinv * z


def _tc_prep(p0, p1, x, w1):
    return pl.pallas_call(
        _prep_body,
        grid=(N // MBLK,),
        in_specs=[
            pl.BlockSpec((MBLK, H), lambda i: (i, 0)),
            pl.BlockSpec((MBLK, H), lambda i: (i, 0)),
            pl.BlockSpec((MBLK, D), lambda i: (i, 0)),
            pl.BlockSpec((D, H), lambda i: (0, 0)),
        ],
        out_specs=[
            pl.BlockSpec((MBLK, 1), lambda i: (i, 0)),
            pl.BlockSpec((MBLK, H), lambda i: (i, 0)),
            pl.BlockSpec((MBLK, H), lambda i: (i, 0)),
        ],
        out_shape=[
            jax.ShapeDtypeStruct((N, 1), _F32),
            jax.ShapeDtypeStruct((N, H), _F32),
            jax.ShapeDtypeStruct((N, H), _F32),
        ],
    )(p0, p1, x, w1)


def _mid_body(q0, q1, z, dinv, b, w, h_o, zn_o, gn_o):
    dv = dinv[...]
    h = jnp.maximum(dv * (q0[...] + q1[...]) + dv * dv * z[...] + b[...], 0.0)
    h_o[...] = h
    zn = jnp.dot(h, w[...], preferred_element_type=_F32)
    zn_o[...] = zn
    gn_o[...] = dv * zn


def _tc_mid(q0, q1, z, dinv, b, wnext):
    return pl.pallas_call(
        _mid_body,
        grid=(N // MBLK,),
        in_specs=[
            pl.BlockSpec((MBLK, H), lambda i: (i, 0)),
            pl.BlockSpec((MBLK, H), lambda i: (i, 0)),
            pl.BlockSpec((MBLK, H), lambda i: (i, 0)),
            pl.BlockSpec((MBLK, 1), lambda i: (i, 0)),
            pl.BlockSpec((1, H), lambda i: (0, 0)),
            pl.BlockSpec((H, H), lambda i: (0, 0)),
        ],
        out_specs=[
            pl.BlockSpec((MBLK, H), lambda i: (i, 0)),
            pl.BlockSpec((MBLK, H), lambda i: (i, 0)),
            pl.BlockSpec((MBLK, H), lambda i: (i, 0)),
        ],
        out_shape=[
            jax.ShapeDtypeStruct((N, H), _F32),
            jax.ShapeDtypeStruct((N, H), _F32),
            jax.ShapeDtypeStruct((N, H), _F32),
        ],
    )(q0, q1, z, dinv, b, wnext)


def _last_body(q0, q1, z, dinv, b, h_o):
    dv = dinv[...]
    h_o[...] = jnp.maximum(
        dv * (q0[...] + q1[...]) + dv * dv * z[...] + b[...], 0.0)


def _tc_last(q0, q1, z, dinv, b):
    return pl.pallas_call(
        _last_body,
        grid=(N // MBLK,),
        in_specs=[
            pl.BlockSpec((MBLK, H), lambda i: (i, 0)),
            pl.BlockSpec((MBLK, H), lambda i: (i, 0)),
            pl.BlockSpec((MBLK, H), lambda i: (i, 0)),
            pl.BlockSpec((MBLK, 1), lambda i: (i, 0)),
            pl.BlockSpec((1, H), lambda i: (0, 0)),
        ],
        out_specs=pl.BlockSpec((MBLK, H), lambda i: (i, 0)),
        out_shape=jax.ShapeDtypeStruct((N, H), _F32),
    )(q0, q1, z, dinv, b)


def _fin_body(pt1, pt2, pt3, batchr, fw1, fb1, fw2, fb2, fw3, fb3,
              out_o, a1, a2, a3):
    w = pl.program_id(0)
    for pt, a in ((pt1, a1), (pt2, a2), (pt3, a3)):
        blk = pt[0]

        @pl.when(w == 0)
        def _(blk=blk, a=a):
            a[...] = blk

        @pl.when(w > 0)
        def _(blk=blk, a=a):
            s = a[:, :H] + blk[:, :H]
            m = jnp.maximum(a[:, H:], blk[:, H:])
            a[...] = jnp.concatenate([s, m], axis=1)

    @pl.when(w == NW - 1)
    def _():
        bv = batchr[...]
        iota_c = lax.broadcasted_iota(jnp.int32, (B, NPAD), 0)
        onehot = (bv == iota_c).astype(_F32)
        cnt = jnp.sum(onehot, axis=1, keepdims=True)
        invc = jnp.where(cnt > 0, 1.0 / jnp.maximum(cnt, 1.0), 0.0)
        zs = jnp.zeros((B, 3 * H), _F32)
        for a in (a1, a2, a3):
            sacc = a[:, :H]
            macc = jnp.where(cnt > 0, a[:, H:], 0.0)
            zs = zs + jnp.concatenate([sacc * invc, macc, sacc], axis=1)
        t1 = jnp.maximum(
            jnp.dot(zs, fw1[...], preferred_element_type=_F32) + fb1[...], 0.0)
        t2 = jnp.maximum(
            jnp.dot(t1, fw2[...], preferred_element_type=_F32) + fb2[...], 0.0)
        logits = jnp.dot(t2, fw3[...], preferred_element_type=_F32) + fb3[...]
        mxl = jnp.max(logits, axis=-1, keepdims=True)
        ex = jnp.exp(logits - mxl)
        lse = mxl + jnp.log(jnp.sum(ex, axis=-1, keepdims=True))
        out_o[...] = logits - lse


def _tc_final(pt1, pt2, pt3, batchr, fw1, fb1, fw2, fb2, fw3, fb3):
    return pl.pallas_call(
        _fin_body,
        grid=(NW,),
        in_specs=[
            pl.BlockSpec((1, B, 2 * H), lambda w: (w, 0, 0)),
            pl.BlockSpec((1, B, 2 * H), lambda w: (w, 0, 0)),
            pl.BlockSpec((1, B, 2 * H), lambda w: (w, 0, 0)),
            pl.BlockSpec((1, NPAD), lambda w: (0, 0)),
            pl.BlockSpec((3 * H, H), lambda w: (0, 0)),
            pl.BlockSpec((1, H), lambda w: (0, 0)),
            pl.BlockSpec((H, H // 2), lambda w: (0, 0)),
            pl.BlockSpec((1, H // 2), lambda w: (0, 0)),
            pl.BlockSpec((H // 2, 128), lambda w: (0, 0)),
            pl.BlockSpec((1, 128), lambda w: (0, 0)),
        ],
        out_specs=pl.BlockSpec((B, 128), lambda w: (0, 0)),
        out_shape=jax.ShapeDtypeStruct((B, 128), _F32),
        scratch_shapes=[
            pltpu.VMEM((B, 2 * H), _F32),
            pltpu.VMEM((B, 2 * H), _F32),
            pltpu.VMEM((B, 2 * H), _F32),
        ],
    )(pt1, pt2, pt3, batchr, fw1, fb1, fw2, fb2, fw3, fb3)


@jax.jit
def kernel(x, edge_index, batch, y, W1, b1, W2, b2, W3, b3,
           fW1, fb1, fW2, fb2, fW3, fb3):
    src = edge_index[0].astype(jnp.int32)
    dst = edge_index[1].astype(jnp.int32)
    batch = batch.astype(jnp.int32)

    srcp = jnp.concatenate(
        [src, jnp.arange(EPAD, dtype=jnp.int32) % jnp.int32(N)])
    dstp = jnp.concatenate(
        [dst, N + jnp.arange(EPAD, dtype=jnp.int32) % (NP - N)])

    degp = _sc_deg(dstp)
    dinv, z1, g1 = _tc_prep(degp[:N], degp[NP:NP + N], x, W1)

    q1 = _sc_conv(g1, srcp, dstp)
    h1, z2, g2 = _tc_mid(q1[:N], q1[NP:NP + N], z1, dinv, b1.reshape(1, H), W2)

    q2 = _sc_conv(g2, srcp, dstp)
    h2, z3, g3 = _tc_mid(q2[:N], q2[NP:NP + N], z2, dinv, b2.reshape(1, H), W3)

    q3 = _sc_conv(g3, srcp, dstp)
    h3 = _tc_last(q3[:N], q3[NP:NP + N], z3, dinv, b3.reshape(1, H))

    p1 = _sc_pool(h1, batch)
    p2 = _sc_pool(h2, batch)
    p3 = _sc_pool(h3, batch)

    batchr = jnp.concatenate(
        [batch, jnp.full((NPAD - N,), B, jnp.int32)]).reshape(1, NPAD)
    fw3p = jnp.pad(fW3, ((0, 0), (0, 128 - C)))
    fb3p = jnp.pad(fb3, (0, 128 - C), constant_values=_NEG).reshape(1, 128)

    out = _tc_final(p1, p2, p3, batchr,
                    fW1, fb1.reshape(1, H),
                    fW2, fb2.reshape(1, H // 2),
                    fw3p, fb3p)
    return out[:, :C]


# speedup vs baseline: 3.3177x; 1.0874x over previous
import dataclasses
import functools

import jax
import jax.numpy as jnp
from jax import lax
from jax.experimental import pallas as pl
from jax.experimental.pallas import tpu as pltpu
from jax.experimental.pallas import tpu_sc as plsc

N = 10000
E = 320000
D = 128
H = 128
B = 128
C = 10

NC = 2
NS = 16
NW = NC * NS
EK = 128
NBLKP = 2560
BPW = NBLKP // NW
EPAD = NBLKP * EK - E
NP = 10240
RPT = NP // NS
ROWS_W = 320
PCH = 16
MBLK = 400
NPAD = 10112

_F32 = jnp.float32
_NEG = -3.0e38

_vec_mesh = plsc.VectorSubcoreMesh(core_axis_name="c", subcore_axis_name="s")

_sc_params = pltpu.CompilerParams()
if "needs_layout_passes" in pltpu.CompilerParams.__dataclass_fields__:
    _sc_params = dataclasses.replace(_sc_params, needs_layout_passes=False)


def _wid():
    return lax.axis_index("s") * NC + lax.axis_index("c")


def _fill(ref, nrows, vec16):
    @pl.loop(0, nrows)
    def _(i):
        for k in range(8):
            ref[i, pl.ds(16 * k, 16)] = vec16


@functools.partial(
    pl.kernel,
    mesh=_vec_mesh,
    out_type=jax.ShapeDtypeStruct((2 * NP, H), _F32),
    scratch_types=[
        pltpu.VMEM_SHARED((NP, H), _F32),
        pltpu.VMEM((EK, H), _F32),
        [pltpu.VMEM((EK,), jnp.int32)] * 4,
        [pltpu.SemaphoreType.DMA] * 4,
        [pltpu.SemaphoreType.DMA] * 2,
    ],
)
def _sc_deg(dstp_hbm, out_hbm, acc_sh, ones_v, dvs, sis, sss):
    c = lax.axis_index("c")
    s = lax.axis_index("s")
    w = _wid()

    _fill(ones_v, EK, jnp.zeros((16,), _F32))

    @pl.loop(0, 5)
    def _(t):
        pltpu.sync_copy(ones_v, acc_sh.at[pl.ds(s * RPT + t * EK, EK)])

    _fill(ones_v, EK, jnp.ones((16,), _F32))

    def idx_slice(t):
        return dstp_hbm.at[pl.ds((w * BPW + t) * EK, EK)]

    for t in range(2):
        pltpu.async_copy(idx_slice(t), dvs[t], sis[t])

    plsc.subcore_barrier()

    @pl.loop(0, BPW // 4)
    def _(i):
        for b in range(4):
            t = 4 * i + b

            @pl.when(t >= 2)
            def _(t=t, b=b):
                pltpu.make_async_copy(
                    ones_v, acc_sh.at[dvs[(b + 2) % 4]], sss[b % 2]).wait()

            pltpu.make_async_copy(idx_slice(t), dvs[b], sis[b]).wait()

            @pl.when(t < BPW - 2)
            def _(t=t, b=b):
                pltpu.async_copy(
                    idx_slice(t + 2), dvs[(b + 2) % 4], sis[(b + 2) % 4])

            pltpu.async_copy(ones_v, acc_sh.at[dvs[b]], sss[b % 2], add=True)

    for b in range(2):
        pltpu.make_async_copy(
            ones_v, acc_sh.at[dvs[(BPW - 2 + b) % 4]], sss[b]).wait()

    plsc.subcore_barrier()
    pltpu.sync_copy(acc_sh.at[pl.ds(s * RPT, RPT)],
                    out_hbm.at[pl.ds(c * NP + s * RPT, RPT)])


@functools.partial(
    pl.kernel,
    mesh=_vec_mesh,
    out_type=jax.ShapeDtypeStruct((2 * NP, H), _F32),
    scratch_types=[
        pltpu.VMEM_SHARED((NP, H), _F32),
        [pltpu.VMEM((EK, H), _F32)] * 2,
        [pltpu.VMEM((EK,), jnp.int32)] * 4,
        [pltpu.VMEM((EK,), jnp.int32)] * 4,
        [pltpu.SemaphoreType.DMA] * 4,
        [pltpu.SemaphoreType.DMA] * 4,
        [pltpu.SemaphoreType.DMA] * 2,
    ],
)
def _sc_conv(g_hbm, srcp_hbm, dstp_hbm, out_hbm, acc_sh, rows, svs, dvs,
             sis, sjs, sgs):
    c = lax.axis_index("c")
    s = lax.axis_index("s")
    w = _wid()

    _fill(rows[0], EK, jnp.zeros((16,), _F32))

    @pl.loop(0, 5)
    def _(t):
        pltpu.sync_copy(rows[0], acc_sh.at[pl.ds(s * RPT + t * EK, EK)])

    def src_slice(t):
        return srcp_hbm.at[pl.ds((w * BPW + t) * EK, EK)]

    def dst_slice(t):
        return dstp_hbm.at[pl.ds((w * BPW + t) * EK, EK)]

    for t in range(2):
        pltpu.async_copy(src_slice(t), svs[t], sis[t])
        pltpu.async_copy(dst_slice(t), dvs[t], sjs[t])
    pltpu.make_async_copy(src_slice(0), svs[0], sis[0]).wait()
    pltpu.async_copy(g_hbm.at[svs[0]], rows[0], sgs[0])

    plsc.subcore_barrier()

    @pl.loop(0, BPW // 4)
    def _(i):
        for b in range(4):
            t = 4 * i + b
            b2 = b % 2

            pltpu.make_async_copy(g_hbm.at[svs[b]], rows[b2], sgs[b2]).wait()

            @pl.when(t < BPW - 1)
            def _(t=t, b=b, b2=b2):
                nb = (b + 1) % 4
                pltpu.make_async_copy(src_slice(t + 1), svs[nb], sis[nb]).wait()
                pltpu.async_copy(g_hbm.at[svs[nb]], rows[1 - b2], sgs[1 - b2])

            @pl.when(t < BPW - 2)
            def _(t=t, b=b):
                nb = (b + 2) % 4
                pltpu.async_copy(src_slice(t + 2), svs[nb], sis[nb])
                pltpu.async_copy(dst_slice(t + 2), dvs[nb], sjs[nb])

            pltpu.make_async_copy(dst_slice(t), dvs[b], sjs[b]).wait()
            pltpu.sync_copy(rows[b2], acc_sh.at[dvs[b]], add=True)

    plsc.subcore_barrier()
    pltpu.sync_copy(acc_sh.at[pl.ds(s * RPT, RPT)],
                    out_hbm.at[pl.ds(c * NP + s * RPT, RPT)])


@functools.partial(
    pl.kernel,
    mesh=_vec_mesh,
    out_type=jax.ShapeDtypeStruct((NW, B, 2 * H), _F32),
    scratch_types=[
        pltpu.VMEM((B, 2 * H), _F32),
        [pltpu.VMEM((PCH, H), _F32)] * 2,
        [pltpu.VMEM((PCH,), jnp.int32)] * 2,
        [pltpu.SemaphoreType.DMA] * 2,
        [pltpu.SemaphoreType.DMA] * 2,
    ],
    compiler_params=_sc_params,
)
def _sc_pool(h_hbm, batch_hbm, out_hbm, acc_v, h_vs, b_vs, shs, sbs):
    w = _wid()
    zero16 = jnp.zeros((16,), _F32)
    neg16 = jnp.full((16,), _NEG, _F32)
    iota16 = lax.broadcasted_iota(jnp.int32, (16,), 0)

    @pl.loop(0, B)
    def _(i):
        for k in range(8):
            acc_v[i, pl.ds(16 * k, 16)] = zero16
            acc_v[i, pl.ds(H + 16 * k, 16)] = neg16

    lo = w * ROWS_W
    nchunks = jnp.minimum(ROWS_W, N - lo) // PCH

    def issue(ci, b):
        off = lo + ci * PCH
        pltpu.async_copy(batch_hbm.at[pl.ds(off, PCH)], b_vs[b], sbs[b])
        pltpu.async_copy(h_hbm.at[pl.ds(off, PCH)], h_vs[b], shs[b])

    def wait(ci, b):
        off = lo + ci * PCH
        pltpu.make_async_copy(
            batch_hbm.at[pl.ds(off, PCH)], b_vs[b], sbs[b]).wait()
        pltpu.make_async_copy(
            h_hbm.at[pl.ds(off, PCH)], h_vs[b], shs[b]).wait()

    for b in range(2):
        @pl.when(b < nchunks)
        def _(b=b):
            issue(b, b)

    @pl.loop(0, ROWS_W // PCH // 2)
    def _(i):
        for b in range(2):
            ci = 2 * i + b

            @pl.when(ci < nchunks)
            def _(ci=ci, b=b):
                wait(ci, b)

                @pl.when(ci + 2 < nchunks)
                def _(ci=ci, b=b):
                    issue(ci + 2, b)

                bvec = b_vs[b][...]
                for r in range(PCH):
                    g = jnp.max(jnp.where(iota16 == r, bvec, 0))
                    for k in range(8):
                        hx = h_vs[b][r, pl.ds(16 * k, 16)]
                        acc_v[g, pl.ds(16 * k, 16)] = (
                            acc_v[g, pl.ds(16 * k, 16)] + hx)
                        acc_v[g, pl.ds(H + 16 * k, 16)] = jnp.maximum(
                            acc_v[g, pl.ds(H + 16 * k, 16)], hx)

    pltpu.sync_copy(acc_v, out_hbm.at[w])


def _prep_body(p0, p1, x, w, dinv_o, z_o, g_o):
    deg = 1.0 + p0[:, 0:1] + p1[:, 0:1]
    dinv = lax.rsqrt(deg)
    z = jnp.dot(x[...], w[...], preferred_element_type=_F32)
    dinv_o[...] = dinv
    z_o[...] = z
    g_o[...] = dinv * z


def _tc_prep(p0, p1, x, w1):
    return pl.pallas_call(
        _prep_body,
        grid=(N // MBLK,),
        in_specs=[
            pl.BlockSpec((MBLK, H), lambda i: (i, 0)),
            pl.BlockSpec((MBLK, H), lambda i: (i, 0)),
            pl.BlockSpec((MBLK, D), lambda i: (i, 0)),
            pl.BlockSpec((D, H), lambda i: (0, 0)),
        ],
        out_specs=[
            pl.BlockSpec((MBLK, 1), lambda i: (i, 0)),
            pl.BlockSpec((MBLK, H), lambda i: (i, 0)),
            pl.BlockSpec((MBLK, H), lambda i: (i, 0)),
        ],
        out_shape=[
            jax.ShapeDtypeStruct((N, 1), _F32),
            jax.ShapeDtypeStruct((N, H), _F32),
            jax.ShapeDtypeStruct((N, H), _F32),
        ],
    )(p0, p1, x, w1)


def _mid_body(q0, q1, z, dinv, b, w, h_o, zn_o, gn_o):
    dv = dinv[...]
    h = jnp.maximum(dv * (q0[...] + q1[...]) + dv * dv * z[...] + b[...], 0.0)
    h_o[...] = h
    zn = jnp.dot(h, w[...], preferred_element_type=_F32)
    zn_o[...] = zn
    gn_o[...] = dv * zn


def _tc_mid(q0, q1, z, dinv, b, wnext):
    return pl.pallas_call(
        _mid_body,
        grid=(N // MBLK,),
        in_specs=[
            pl.BlockSpec((MBLK, H), lambda i: (i, 0)),
            pl.BlockSpec((MBLK, H), lambda i: (i, 0)),
            pl.BlockSpec((MBLK, H), lambda i: (i, 0)),
            pl.BlockSpec((MBLK, 1), lambda i: (i, 0)),
            pl.BlockSpec((1, H), lambda i: (0, 0)),
            pl.BlockSpec((H, H), lambda i: (0, 0)),
        ],
        out_specs=[
            pl.BlockSpec((MBLK, H), lambda i: (i, 0)),
            pl.BlockSpec((MBLK, H), lambda i: (i, 0)),
            pl.BlockSpec((MBLK, H), lambda i: (i, 0)),
        ],
        out_shape=[
            jax.ShapeDtypeStruct((N, H), _F32),
            jax.ShapeDtypeStruct((N, H), _F32),
            jax.ShapeDtypeStruct((N, H), _F32),
        ],
    )(q0, q1, z, dinv, b, wnext)


def _last_body(q0, q1, z, dinv, b, h_o):
    dv = dinv[...]
    h_o[...] = jnp.maximum(
        dv * (q0[...] + q1[...]) + dv * dv * z[...] + b[...], 0.0)


def _tc_last(q0, q1, z, dinv, b):
    return pl.pallas_call(
        _last_body,
        grid=(N // MBLK,),
        in_specs=[
            pl.BlockSpec((MBLK, H), lambda i: (i, 0)),
            pl.BlockSpec((MBLK, H), lambda i: (i, 0)),
            pl.BlockSpec((MBLK, H), lambda i: (i, 0)),
            pl.BlockSpec((MBLK, 1), lambda i: (i, 0)),
            pl.BlockSpec((1, H), lambda i: (0, 0)),
        ],
        out_specs=pl.BlockSpec((MBLK, H), lambda i: (i, 0)),
        out_shape=jax.ShapeDtypeStruct((N, H), _F32),
    )(q0, q1, z, dinv, b)


def _fin_body(pt1, pt2, pt3, batchr, fw1, fb1, fw2, fb2, fw3, fb3,
              out_o, a1, a2, a3):
    w = pl.program_id(0)
    for pt, a in ((pt1, a1), (pt2, a2), (pt3, a3)):
        blk = pt[0]

        @pl.when(w == 0)
        def _(blk=blk, a=a):
            a[...] = blk

        @pl.when(w > 0)
        def _(blk=blk, a=a):
            s = a[:, :H] + blk[:, :H]
            m = jnp.maximum(a[:, H:], blk[:, H:])
            a[...] = jnp.concatenate([s, m], axis=1)

    @pl.when(w == NW - 1)
    def _():
        bv = batchr[...]
        iota_c = lax.broadcasted_iota(jnp.int32, (B, NPAD), 0)
        onehot = (bv == iota_c).astype(_F32)
        cnt = jnp.sum(onehot, axis=1, keepdims=True)
        invc = jnp.where(cnt > 0, 1.0 / jnp.maximum(cnt, 1.0), 0.0)
        zs = jnp.zeros((B, 3 * H), _F32)
        for a in (a1, a2, a3):
            sacc = a[:, :H]
            macc = jnp.where(cnt > 0, a[:, H:], 0.0)
            zs = zs + jnp.concatenate([sacc * invc, macc, sacc], axis=1)
        t1 = jnp.maximum(
            jnp.dot(zs, fw1[...], preferred_element_type=_F32) + fb1[...], 0.0)
        t2 = jnp.maximum(
            jnp.dot(t1, fw2[...], preferred_element_type=_F32) + fb2[...], 0.0)
        logits = jnp.dot(t2, fw3[...], preferred_element_type=_F32) + fb3[...]
        mxl = jnp.max(logits, axis=-1, keepdims=True)
        ex = jnp.exp(logits - mxl)
        lse = mxl + jnp.log(jnp.sum(ex, axis=-1, keepdims=True))
        out_o[...] = logits - lse


def _tc_final(pt1, pt2, pt3, batchr, fw1, fb1, fw2, fb2, fw3, fb3):
    return pl.pallas_call(
        _fin_body,
        grid=(NW,),
        in_specs=[
            pl.BlockSpec((1, B, 2 * H), lambda w: (w, 0, 0)),
            pl.BlockSpec((1, B, 2 * H), lambda w: (w, 0, 0)),
            pl.BlockSpec((1, B, 2 * H), lambda w: (w, 0, 0)),
            pl.BlockSpec((1, NPAD), lambda w: (0, 0)),
            pl.BlockSpec((3 * H, H), lambda w: (0, 0)),
            pl.BlockSpec((1, H), lambda w: (0, 0)),
            pl.BlockSpec((H, H // 2), lambda w: (0, 0)),
            pl.BlockSpec((1, H // 2), lambda w: (0, 0)),
            pl.BlockSpec((H // 2, 128), lambda w: (0, 0)),
            pl.BlockSpec((1, 128), lambda w: (0, 0)),
        ],
        out_specs=pl.BlockSpec((B, 128), lambda w: (0, 0)),
        out_shape=jax.ShapeDtypeStruct((B, 128), _F32),
        scratch_shapes=[
            pltpu.VMEM((B, 2 * H), _F32),
            pltpu.VMEM((B, 2 * H), _F32),
            pltpu.VMEM((B, 2 * H), _F32),
        ],
    )(pt1, pt2, pt3, batchr, fw1, fb1, fw2, fb2, fw3, fb3)


@jax.jit
def kernel(x, edge_index, batch, y, W1, b1, W2, b2, W3, b3,
           fW1, fb1, fW2, fb2, fW3, fb3):
    src = edge_index[0].astype(jnp.int32)
    dst = edge_index[1].astype(jnp.int32)
    batch = batch.astype(jnp.int32)

    srcp = jnp.concatenate(
        [src, jnp.arange(EPAD, dtype=jnp.int32) % jnp.int32(N)])
    dstp = jnp.concatenate(
        [dst, N + jnp.arange(EPAD, dtype=jnp.int32) % (NP - N)])

    degp = _sc_deg(dstp)
    dinv, z1, g1 = _tc_prep(degp[:N], degp[NP:NP + N], x, W1)

    q1 = _sc_conv(g1, srcp, dstp)
    h1, z2, g2 = _tc_mid(q1[:N], q1[NP:NP + N], z1, dinv, b1.reshape(1, H), W2)

    q2 = _sc_conv(g2, srcp, dstp)
    h2, z3, g3 = _tc_mid(q2[:N], q2[NP:NP + N], z2, dinv, b2.reshape(1, H), W3)

    q3 = _sc_conv(g3, srcp, dstp)
    h3 = _tc_last(q3[:N], q3[NP:NP + N], z3, dinv, b3.reshape(1, H))

    p1 = _sc_pool(h1, batch)
    p2 = _sc_pool(h2, batch)
    p3 = _sc_pool(h3, batch)

    batchr = jnp.concatenate(
        [batch, jnp.full((NPAD - N,), B, jnp.int32)]).reshape(1, NPAD)
    fw3p = jnp.pad(fW3, ((0, 0), (0, 128 - C)))
    fb3p = jnp.pad(fb3, (0, 128 - C), constant_values=_NEG).reshape(1, 128)

    out = _tc_final(p1, p2, p3, batchr,
                    fW1, fb1.reshape(1, H),
                    fW2, fb2.reshape(1, H // 2),
                    fw3p, fb3p)
    return out[:, :C]
